# baseline (device time: 35665 ns/iter reference)
import jax
import jax.numpy as jnp
from jax import lax
from jax.experimental import pallas as pl
from jax.experimental.pallas import tpu as pltpu

N_DEV = 4


def kernel(x, Wq, K_ext, V_ext, Wo):
    B, Sq, D = x.shape
    Hq_loc, Dh = K_ext.shape[2:]
    d_loc = Hq_loc * Dh

    def body(x_ref, wq_ref, k_ref, v_ref, wo_ref, out_ref,
             comm_ref, send_sems, recv_sems):
        my = lax.axis_index("i")
        left = lax.rem(my + N_DEV - 1, N_DEV)
        right = lax.rem(my + 1, N_DEV)

        barrier_sem = pltpu.get_barrier_semaphore()
        for nbr in (left, right):
            pl.semaphore_signal(barrier_sem, inc=1, device_id=(nbr,),
                                device_id_type=pl.DeviceIdType.MESH)
        pl.semaphore_wait(barrier_sem, 2)

        wq_loc = wq_ref[:, pl.ds(my * d_loc, d_loc)]
        wo_loc = wo_ref[pl.ds(my * d_loc, d_loc), :]
        for b in range(B):
            q = jnp.dot(x_ref[b], wq_loc,
                        preferred_element_type=jnp.float32)
            ctx_parts = []
            for h in range(Hq_loc):
                qh = q[:, h * Dh:(h + 1) * Dh]
                kh = k_ref[b, :, h, :]
                vh = v_ref[b, :, h, :]
                s = lax.dot_general(
                    qh, kh, (((1,), (1,)), ((), ())),
                    preferred_element_type=jnp.float32) * 0.125
                s_max = jnp.max(s, axis=-1, keepdims=True)
                w = jnp.exp(s - s_max)
                w = w / jnp.sum(w, axis=-1, keepdims=True)
                ctx_parts.append(jnp.dot(w, vh,
                                         preferred_element_type=jnp.float32))
            ctx = jnp.concatenate(ctx_parts, axis=-1)
            pb = jnp.dot(ctx, wo_loc,
                         preferred_element_type=jnp.float32)
            out_ref[b] = pb
            comm_ref[0, b] = pb

        for h in range(N_DEV - 1):
            rdma = pltpu.make_async_remote_copy(
                src_ref=comm_ref.at[h],
                dst_ref=comm_ref.at[h + 1],
                send_sem=send_sems.at[h],
                recv_sem=recv_sems.at[h],
                device_id=(right,),
                device_id_type=pl.DeviceIdType.MESH,
            )
            rdma.start()
            rdma.wait()
            out_ref[:, :, :] = out_ref[:, :, :] + comm_ref[h + 1]

    return pl.pallas_call(
        body,
        out_shape=jax.ShapeDtypeStruct((B, Sq, D), jnp.float32),
        in_specs=[pl.BlockSpec(memory_space=pltpu.VMEM)] * 5,
        out_specs=pl.BlockSpec(memory_space=pltpu.VMEM),
        scratch_shapes=[
            pltpu.VMEM((N_DEV, B, Sq, D), jnp.float32),
            pltpu.SemaphoreType.DMA((N_DEV - 1,)),
            pltpu.SemaphoreType.DMA((N_DEV - 1,)),
        ],
        compiler_params=pltpu.CompilerParams(collective_id=0),
    )(x, Wq, K_ext, V_ext, Wo)


# device time: 21256 ns/iter; 1.6779x vs baseline; 1.6779x over previous
import jax
import jax.numpy as jnp
from jax import lax
from jax.experimental import pallas as pl
from jax.experimental.pallas import tpu as pltpu

N_DEV = 4


def kernel(x, Wq, K_ext, V_ext, Wo):
    B, Sq, D = x.shape
    Hq_loc, Dh = K_ext.shape[2:]
    d_loc = Hq_loc * Dh
    R = B * Sq

    def body(x_ref, wq_ref, k_ref, v_ref, wo_ref, out_ref,
             comm_ref, acc_ref, send_sems, recv_sems):
        my = lax.axis_index("i")

        barrier_sem = pltpu.get_barrier_semaphore()
        for o in range(1, N_DEV):
            pl.semaphore_signal(barrier_sem, inc=1,
                                device_id=(lax.rem(my + o, N_DEV),),
                                device_id_type=pl.DeviceIdType.MESH)
        pl.semaphore_wait(barrier_sem, N_DEV - 1)

        wq_loc = wq_ref[:, pl.ds(my * d_loc, d_loc)]
        for b in range(B):
            q = jnp.dot(x_ref[b], wq_loc,
                        preferred_element_type=jnp.float32)
            ctx_parts = []
            for h in range(Hq_loc):
                qh = q[:, h * Dh:(h + 1) * Dh]
                kh = k_ref[b, :, h, :]
                vh = v_ref[b, :, h, :]
                s = lax.dot_general(
                    qh, kh, (((1,), (1,)), ((), ())),
                    preferred_element_type=jnp.float32) * 0.125
                s_max = jnp.max(s, axis=-1, keepdims=True)
                w = jnp.exp(s - s_max)
                w = w / jnp.sum(w, axis=-1, keepdims=True)
                ctx_parts.append(jnp.dot(w, vh,
                                         preferred_element_type=jnp.float32))
            comm_ref[0, pl.ds(b * Sq, Sq), :] = jnp.concatenate(
                ctx_parts, axis=-1)

        rdmas = {}
        for o in (2, 1, 3):
            rdmas[o] = pltpu.make_async_remote_copy(
                src_ref=comm_ref.at[0],
                dst_ref=comm_ref.at[o],
                send_sem=send_sems.at[o - 1],
                recv_sem=recv_sems.at[o - 1],
                device_id=(lax.rem(my + o, N_DEV),),
                device_id_type=pl.DeviceIdType.MESH,
            )
            rdmas[o].start()

        acc_ref[:, :] = jnp.dot(
            comm_ref[0], wo_ref[pl.ds(my * d_loc, d_loc), :],
            preferred_element_type=jnp.float32)

        for k in (1, 3, 2):
            rdmas[k].wait_recv()
            origin = lax.rem(my - k + N_DEV, N_DEV)
            acc_ref[:, :] = acc_ref[:, :] + jnp.dot(
                comm_ref[k], wo_ref[pl.ds(origin * d_loc, d_loc), :],
                preferred_element_type=jnp.float32)

        for o in (1, 2, 3):
            rdmas[o].wait_send()

        for b in range(B):
            out_ref[b] = acc_ref[pl.ds(b * Sq, Sq), :]

    return pl.pallas_call(
        body,
        out_shape=jax.ShapeDtypeStruct((B, Sq, D), jnp.float32),
        in_specs=[pl.BlockSpec(memory_space=pltpu.VMEM)] * 5,
        out_specs=pl.BlockSpec(memory_space=pltpu.VMEM),
        scratch_shapes=[
            pltpu.VMEM((N_DEV, R, d_loc), jnp.float32),
            pltpu.VMEM((R, D), jnp.float32),
            pltpu.SemaphoreType.DMA((N_DEV - 1,)),
            pltpu.SemaphoreType.DMA((N_DEV - 1,)),
        ],
        compiler_params=pltpu.CompilerParams(collective_id=0),
    )(x, Wq, K_ext, V_ext, Wo)


# device time: 12945 ns/iter; 2.7551x vs baseline; 1.6420x over previous
import jax
import jax.numpy as jnp
from jax import lax
from jax.experimental import pallas as pl
from jax.experimental.pallas import tpu as pltpu

N_DEV = 4


def kernel(x, Wq, K_ext, V_ext, Wo):
    B, Sq, D = x.shape
    Hq_loc, Dh = K_ext.shape[2:]
    d_loc = Hq_loc * Dh
    R = B * Sq
    Dm = Wq.shape[1]

    def body(q_hbm, kvt_hbm, wo_hbm, out_hbm,
             q_v, kvt_v, wo_v, q16_v, kvt16_v, wo16_v, comm_ref, acc_ref,
             dma_sems, send_sems, recv_sems):
        my = lax.axis_index("i")

        cp_q = pltpu.make_async_copy(q_hbm, q_v, dma_sems.at[0])
        cp_kvt = pltpu.make_async_copy(kvt_hbm, kvt_v, dma_sems.at[1])
        cp_wo = pltpu.make_async_copy(wo_hbm, wo_v, dma_sems.at[2])
        for cp in (cp_q, cp_kvt, cp_wo):
            cp.start()

        barrier_sem = pltpu.get_barrier_semaphore()
        for o in range(1, N_DEV):
            pl.semaphore_signal(barrier_sem, inc=1,
                                device_id=(lax.rem(my + o, N_DEV),),
                                device_id_type=pl.DeviceIdType.MESH)
        pl.semaphore_wait(barrier_sem, N_DEV - 1)

        cp_q.wait()
        cp_kvt.wait()
        q16_v[:, :] = q_v[:, :].astype(jnp.bfloat16)
        kvt16_v[...] = kvt_v[...].astype(jnp.bfloat16)

        for b in range(B):
            for h in range(Hq_loc):
                qh = q16_v[pl.ds(b * Sq, Sq), pl.ds(h * Dh, Dh)]
                kth = kvt16_v[0, b, h]
                vth = kvt16_v[1, b, h]
                s = jnp.dot(qh, kth,
                            preferred_element_type=jnp.float32) * 0.125
                s_max = jnp.max(s, axis=-1, keepdims=True)
                w = jnp.exp(s - s_max)
                w = w / jnp.sum(w, axis=-1, keepdims=True)
                ctx_h = lax.dot_general(
                    w.astype(jnp.bfloat16), vth, (((1,), (1,)), ((), ())),
                    preferred_element_type=jnp.float32)
                comm_ref[0, pl.ds(b * Sq, Sq), pl.ds(h * Dh, Dh)] = (
                    ctx_h.astype(jnp.bfloat16))

        rdmas = {}
        for o in (2, 1, 3):
            rdmas[o] = pltpu.make_async_remote_copy(
                src_ref=comm_ref.at[0],
                dst_ref=comm_ref.at[o],
                send_sem=send_sems.at[o - 1],
                recv_sem=recv_sems.at[o - 1],
                device_id=(lax.rem(my + o, N_DEV),),
                device_id_type=pl.DeviceIdType.MESH,
            )
            rdmas[o].start()

        cp_wo.wait()
        wo16_v[:, :] = wo_v[:, :].astype(jnp.bfloat16)
        acc_ref[:, :] = jnp.dot(
            comm_ref[0], wo16_v[pl.ds(my * d_loc, d_loc), :],
            preferred_element_type=jnp.float32)

        for k in (1, 3, 2):
            rdmas[k].wait_recv()
            origin = lax.rem(my - k + N_DEV, N_DEV)
            acc_ref[:, :] = acc_ref[:, :] + jnp.dot(
                comm_ref[k], wo16_v[pl.ds(origin * d_loc, d_loc), :],
                preferred_element_type=jnp.float32)

        for o in (1, 2, 3):
            rdmas[o].wait_send()

        cp_out = pltpu.make_async_copy(acc_ref, out_hbm, dma_sems.at[3])
        cp_out.start()
        cp_out.wait()

    my_out = lax.axis_index("i")
    wq_loc = lax.dynamic_slice_in_dim(Wq, my_out * d_loc, d_loc, axis=1)
    q = jnp.dot(x.reshape(R, D), wq_loc,
                preferred_element_type=jnp.float32)
    kt = jnp.transpose(K_ext, (0, 2, 3, 1))
    vt = jnp.transpose(V_ext, (0, 2, 3, 1))
    kvt = jnp.stack([kt, vt])

    out = pl.pallas_call(
        body,
        out_shape=jax.ShapeDtypeStruct((R, D), jnp.float32),
        in_specs=[pl.BlockSpec(memory_space=pl.ANY)] * 3,
        out_specs=pl.BlockSpec(memory_space=pl.ANY),
        scratch_shapes=[
            pltpu.VMEM((R, d_loc), jnp.float32),
            pltpu.VMEM((2, B, Hq_loc, Dh, Sq), jnp.float32),
            pltpu.VMEM((Dm, D), jnp.float32),
            pltpu.VMEM((R, d_loc), jnp.bfloat16),
            pltpu.VMEM((2, B, Hq_loc, Dh, Sq), jnp.bfloat16),
            pltpu.VMEM((Dm, D), jnp.bfloat16),
            pltpu.VMEM((N_DEV, R, d_loc), jnp.bfloat16),
            pltpu.VMEM((R, D), jnp.float32),
            pltpu.SemaphoreType.DMA((4,)),
            pltpu.SemaphoreType.DMA((N_DEV - 1,)),
            pltpu.SemaphoreType.DMA((N_DEV - 1,)),
        ],
        compiler_params=pltpu.CompilerParams(collective_id=0),
    )(q, kvt, Wo)
    return out.reshape(B, Sq, D)


# device time: 12681 ns/iter; 2.8125x vs baseline; 1.0208x over previous
import jax
import jax.numpy as jnp
from jax import lax
from jax.experimental import pallas as pl
from jax.experimental.pallas import tpu as pltpu

N_DEV = 4


def kernel(x, Wq, K_ext, V_ext, Wo):
    B, Sq, D = x.shape
    Hq_loc, Dh = K_ext.shape[2:]
    d_loc = Hq_loc * Dh
    R = B * Sq
    Dm = Wq.shape[1]

    def body(q_hbm, kvt_hbm, wo_hbm, out_hbm,
             q_v, kvt_v, wo_v, q16_v, kvt16_v, wo16_v, comm_ref, acc_ref,
             dma_sems, send_sems, recv_sems):
        my = lax.axis_index("i")

        cp_q = pltpu.make_async_copy(q_hbm, q_v, dma_sems.at[0])
        cp_kvt = pltpu.make_async_copy(kvt_hbm, kvt_v, dma_sems.at[1])
        cp_wo = pltpu.make_async_copy(wo_hbm, wo_v, dma_sems.at[2])
        for cp in (cp_q, cp_kvt, cp_wo):
            cp.start()

        cp_q.wait()
        cp_kvt.wait()
        q16_v[:, :] = q_v[:, :].astype(jnp.bfloat16)
        kvt16_v[...] = kvt_v[...].astype(jnp.bfloat16)

        for b in range(B):
            for h in range(Hq_loc):
                qh = q16_v[pl.ds(b * Sq, Sq), pl.ds(h * Dh, Dh)]
                kth = kvt16_v[b, h, :Dh]
                vth = kvt16_v[b, h, Dh:]
                s = jnp.dot(qh, kth,
                            preferred_element_type=jnp.float32)
                w = jnp.exp(s)
                w_sum = jnp.sum(w, axis=-1, keepdims=True)
                ctx_h = lax.dot_general(
                    w.astype(jnp.bfloat16), vth, (((1,), (1,)), ((), ())),
                    preferred_element_type=jnp.float32) / w_sum
                comm_ref[0, pl.ds(b * Sq, Sq), pl.ds(h * Dh, Dh)] = (
                    ctx_h.astype(jnp.bfloat16))

        barrier_sem = pltpu.get_barrier_semaphore()
        for o in range(1, N_DEV):
            pl.semaphore_signal(barrier_sem, inc=1,
                                device_id=(lax.rem(my + o, N_DEV),),
                                device_id_type=pl.DeviceIdType.MESH)
        pl.semaphore_wait(barrier_sem, N_DEV - 1)

        rdmas = {}
        for o in (2, 1, 3):
            rdmas[o] = pltpu.make_async_remote_copy(
                src_ref=comm_ref.at[0],
                dst_ref=comm_ref.at[o],
                send_sem=send_sems.at[o - 1],
                recv_sem=recv_sems.at[o - 1],
                device_id=(lax.rem(my + o, N_DEV),),
                device_id_type=pl.DeviceIdType.MESH,
            )
            rdmas[o].start()

        cp_wo.wait()
        wo16_v[:, :] = wo_v[:, :].astype(jnp.bfloat16)
        acc_ref[:, :] = jnp.dot(
            comm_ref[0], wo16_v[pl.ds(my * d_loc, d_loc), :],
            preferred_element_type=jnp.float32)

        for k in (1, 3, 2):
            rdmas[k].wait_recv()
            origin = lax.rem(my - k + N_DEV, N_DEV)
            acc_ref[:, :] = acc_ref[:, :] + jnp.dot(
                comm_ref[k], wo16_v[pl.ds(origin * d_loc, d_loc), :],
                preferred_element_type=jnp.float32)

        for o in (1, 2, 3):
            rdmas[o].wait_send()

        cp_out = pltpu.make_async_copy(acc_ref, out_hbm, dma_sems.at[3])
        cp_out.start()
        cp_out.wait()

    my_out = lax.axis_index("i")
    wq_loc = lax.dynamic_slice_in_dim(Wq, my_out * d_loc, d_loc, axis=1)
    q = jnp.dot(x.reshape(R, D), wq_loc,
                preferred_element_type=jnp.float32) * 0.125
    kt = jnp.transpose(K_ext, (0, 2, 3, 1))
    vt = jnp.transpose(V_ext, (0, 2, 3, 1))
    kvt = jnp.concatenate([kt, vt], axis=2)

    out = pl.pallas_call(
        body,
        out_shape=jax.ShapeDtypeStruct((R, D), jnp.float32),
        in_specs=[pl.BlockSpec(memory_space=pl.ANY)] * 3,
        out_specs=pl.BlockSpec(memory_space=pl.ANY),
        scratch_shapes=[
            pltpu.VMEM((R, d_loc), jnp.float32),
            pltpu.VMEM((B, Hq_loc, 2 * Dh, Sq), jnp.float32),
            pltpu.VMEM((Dm, D), jnp.float32),
            pltpu.VMEM((R, d_loc), jnp.bfloat16),
            pltpu.VMEM((B, Hq_loc, 2 * Dh, Sq), jnp.bfloat16),
            pltpu.VMEM((Dm, D), jnp.bfloat16),
            pltpu.VMEM((N_DEV, R, d_loc), jnp.bfloat16),
            pltpu.VMEM((R, D), jnp.float32),
            pltpu.SemaphoreType.DMA((4,)),
            pltpu.SemaphoreType.DMA((N_DEV - 1,)),
            pltpu.SemaphoreType.DMA((N_DEV - 1,)),
        ],
        compiler_params=pltpu.CompilerParams(collective_id=0),
    )(q, kvt, Wo)
    return out.reshape(B, Sq, D)
